# baseline (speedup 1.0000x reference)
import dataclasses
import functools

import jax
import jax.numpy as jnp
from jax import lax
from jax.experimental import pallas as pl
from jax.experimental.pallas import tpu as pltpu
from jax.experimental.pallas import tpu_sc as plsc

_N_CORES = 2
_N_SUBCORES = 16
_COLS_PER_TEC = 8
_EDGE_CHUNK = 1280


def _edge_diff_sc(table_t, idx_i, idx_j):
    n_feat, n_nodes = table_t.shape
    e_total = idx_i.shape[0]
    per_core = e_total // _N_CORES
    n_chunks = per_core // _EDGE_CHUNK
    n_pad = 2 * ((n_chunks + 1) // 2)
    mesh = plsc.VectorSubcoreMesh(core_axis_name="c", subcore_axis_name="s")
    cp = pltpu.CompilerParams()
    if "needs_layout_passes" in pltpu.CompilerParams.__dataclass_fields__:
        cp = dataclasses.replace(cp, needs_layout_passes=False)

    @functools.partial(
        pl.kernel,
        mesh=mesh,
        compiler_params=cp,
        out_type=jax.ShapeDtypeStruct((n_feat, e_total), jnp.float32),
        scratch_types=(
            [pltpu.VMEM((_COLS_PER_TEC, n_nodes), jnp.float32)]
            + [pltpu.VMEM((_EDGE_CHUNK,), jnp.int32) for _ in range(4)]
            + [pltpu.VMEM((_COLS_PER_TEC, _EDGE_CHUNK), jnp.float32) for _ in range(2)]
            + [pltpu.SemaphoreType.DMA for _ in range(6)]
        ),
    )
    def k(tab_hbm, ii_hbm, jj_hbm, out_hbm, tabv, *scratch):
        ii = scratch[0:2]
        jj = scratch[2:4]
        ov = scratch[4:6]
        sii = scratch[6:8]
        sjj = scratch[8:10]
        sw = scratch[10:12]
        core = lax.axis_index("c")
        sub = lax.axis_index("s")
        ebase = core * per_core
        cbase = sub * _COLS_PER_TEC
        pltpu.sync_copy(tab_hbm.at[pl.ds(cbase, _COLS_PER_TEC)], tabv)
        rowsel = [jnp.full((16,), cc, jnp.int32) for cc in range(_COLS_PER_TEC)]

        def issue_idx(c, par):
            off = ebase + c * _EDGE_CHUNK
            pltpu.async_copy(ii_hbm.at[pl.ds(off, _EDGE_CHUNK)], ii[par], sii[par])
            pltpu.async_copy(jj_hbm.at[pl.ds(off, _EDGE_CHUNK)], jj[par], sjj[par])

        def wait_idx(par):
            pltpu.make_async_copy(
                ii_hbm.at[pl.ds(0, _EDGE_CHUNK)], ii[par], sii[par]).wait()
            pltpu.make_async_copy(
                jj_hbm.at[pl.ds(0, _EDGE_CHUNK)], jj[par], sjj[par]).wait()

        def issue_write(c, par):
            pltpu.async_copy(
                ov[par],
                out_hbm.at[pl.ds(cbase, _COLS_PER_TEC),
                           pl.ds(ebase + c * _EDGE_CHUNK, _EDGE_CHUNK)],
                sw[par])

        def wait_write(par):
            pltpu.make_async_copy(
                ov[par],
                out_hbm.at[pl.ds(0, _COLS_PER_TEC), pl.ds(0, _EDGE_CHUNK)],
                sw[par]).wait()

        issue_idx(0, 0)

        @pl.loop(0, n_pad // 2)
        def _(t):
            for par in (0, 1):
                c = t * 2 + par

                @pl.when(c + 1 < n_chunks)
                def _():
                    issue_idx(c + 1, 1 - par)

                @pl.when(jnp.logical_and(c >= 2, c - 2 < n_chunks))
                def _():
                    wait_write(par)

                @pl.when(c < n_chunks)
                def _():
                    wait_idx(par)

                    @plsc.parallel_loop(0, _EDGE_CHUNK // 16, unroll=8)
                    def _(s):
                        sl = pl.ds(s * 16, 16)
                        i16 = ii[par][sl]
                        j16 = jj[par][sl]
                        for cc in range(_COLS_PER_TEC):
                            vi = plsc.load_gather(tabv, [rowsel[cc], i16])
                            vj = plsc.load_gather(tabv, [rowsel[cc], j16])
                            ov[par][cc, sl] = vi - vj

                    issue_write(c, par)

        if n_pad - 2 < n_chunks:
            wait_write((n_pad - 2) % 2)
        if n_pad - 1 < n_chunks:
            wait_write((n_pad - 1) % 2)

    return k(table_t, idx_i, idx_j)


def _fill_body(w_ref, o_ref):
    o_ref[...] = jnp.broadcast_to(w_ref[...], o_ref.shape)


def _rm_weights_tc(w_col, e_total):
    blk = 2560
    return pl.pallas_call(
        _fill_body,
        grid=(e_total // blk,),
        in_specs=[pl.BlockSpec((w_col.shape[0], 1), lambda i: (0, 0))],
        out_specs=pl.BlockSpec((w_col.shape[0], blk), lambda i: (0, i)),
        out_shape=jax.ShapeDtypeStruct((w_col.shape[0], e_total), jnp.float32),
    )(w_col)


def kernel(particles, weights, edges):
    n, p, d = particles.shape
    e_total = edges.shape[1]
    w = weights.shape[1]
    table_t = particles.transpose(1, 2, 0).reshape(p * d, n)
    idx = edges.astype(jnp.int32)
    out_t = _edge_diff_sc(table_t, idx[0], idx[1])
    ratios = out_t.reshape(p, d, e_total).transpose(2, 0, 1)
    w_col = weights[0, :].reshape(w, 1)
    rm_weights = _rm_weights_tc(w_col, e_total).transpose(1, 0)
    return ratios, rm_weights


# speedup vs baseline: 1.2249x; 1.2249x over previous
import dataclasses
import functools

import jax
import jax.numpy as jnp
from jax import lax
from jax.experimental import pallas as pl
from jax.experimental.pallas import tpu as pltpu
from jax.experimental.pallas import tpu_sc as plsc

_N_CORES = 2
_N_SUBCORES = 16
_COLS_PER_TEC = 8
_EDGE_CHUNK = 1280


def _edge_diff_sc(table_t, idx_i, idx_j):
    n_feat, n_nodes = table_t.shape
    e_total = idx_i.shape[0]
    per_core = e_total // _N_CORES
    n_chunks = per_core // _EDGE_CHUNK
    n_pad = 2 * ((n_chunks + 1) // 2)
    mesh = plsc.VectorSubcoreMesh(core_axis_name="c", subcore_axis_name="s")
    cp = pltpu.CompilerParams()
    if "needs_layout_passes" in pltpu.CompilerParams.__dataclass_fields__:
        cp = dataclasses.replace(cp, needs_layout_passes=False)

    @functools.partial(
        pl.kernel,
        mesh=mesh,
        compiler_params=cp,
        out_type=jax.ShapeDtypeStruct((n_feat, e_total), jnp.float32),
        scratch_types=(
            [pltpu.VMEM((_COLS_PER_TEC, n_nodes), jnp.float32)]
            + [pltpu.VMEM((_EDGE_CHUNK,), jnp.int32) for _ in range(4)]
            + [pltpu.VMEM((_COLS_PER_TEC, _EDGE_CHUNK), jnp.float32) for _ in range(2)]
            + [pltpu.SemaphoreType.DMA for _ in range(6)]
        ),
    )
    def k(tab_hbm, ii_hbm, jj_hbm, out_hbm, tabv, *scratch):
        ii = scratch[0:2]
        jj = scratch[2:4]
        ov = scratch[4:6]
        sii = scratch[6:8]
        sjj = scratch[8:10]
        sw = scratch[10:12]
        core = lax.axis_index("c")
        sub = lax.axis_index("s")
        ebase = core * per_core
        cbase = sub * _COLS_PER_TEC
        pltpu.sync_copy(tab_hbm.at[pl.ds(cbase, _COLS_PER_TEC)], tabv)
        rowsel = [jnp.full((16,), cc, jnp.int32) for cc in range(_COLS_PER_TEC)]

        def issue_idx(c, par):
            off = ebase + c * _EDGE_CHUNK
            pltpu.async_copy(ii_hbm.at[pl.ds(off, _EDGE_CHUNK)], ii[par], sii[par])
            pltpu.async_copy(jj_hbm.at[pl.ds(off, _EDGE_CHUNK)], jj[par], sjj[par])

        def wait_idx(par):
            pltpu.make_async_copy(
                ii_hbm.at[pl.ds(0, _EDGE_CHUNK)], ii[par], sii[par]).wait()
            pltpu.make_async_copy(
                jj_hbm.at[pl.ds(0, _EDGE_CHUNK)], jj[par], sjj[par]).wait()

        def issue_write(c, par):
            pltpu.async_copy(
                ov[par],
                out_hbm.at[pl.ds(cbase, _COLS_PER_TEC),
                           pl.ds(ebase + c * _EDGE_CHUNK, _EDGE_CHUNK)],
                sw[par])

        def wait_write(par):
            pltpu.make_async_copy(
                ov[par],
                out_hbm.at[pl.ds(0, _COLS_PER_TEC), pl.ds(0, _EDGE_CHUNK)],
                sw[par]).wait()

        issue_idx(0, 0)

        @pl.loop(0, n_pad // 2)
        def _(t):
            for par in (0, 1):
                c = t * 2 + par

                @pl.when(c + 1 < n_chunks)
                def _():
                    issue_idx(c + 1, 1 - par)

                @pl.when(jnp.logical_and(c >= 2, c - 2 < n_chunks))
                def _():
                    wait_write(par)

                @pl.when(c < n_chunks)
                def _():
                    wait_idx(par)

                    @plsc.parallel_loop(0, _EDGE_CHUNK // 16, unroll=2)
                    def _(s):
                        sl = pl.ds(s * 16, 16)
                        i16 = ii[par][sl]
                        j16 = jj[par][sl]
                        for cc in range(_COLS_PER_TEC):
                            vi = plsc.load_gather(tabv, [rowsel[cc], i16])
                            vj = plsc.load_gather(tabv, [rowsel[cc], j16])
                            ov[par][cc, sl] = vi - vj

                    issue_write(c, par)

        if n_pad - 2 < n_chunks:
            wait_write((n_pad - 2) % 2)
        if n_pad - 1 < n_chunks:
            wait_write((n_pad - 1) % 2)

    return k(table_t, idx_i, idx_j)


def _fill_body(w_ref, o_ref):
    o_ref[...] = jnp.broadcast_to(w_ref[...], o_ref.shape)


def _rm_weights_tc(w_col, e_total):
    blk = 2560
    return pl.pallas_call(
        _fill_body,
        grid=(e_total // blk,),
        in_specs=[pl.BlockSpec((w_col.shape[0], 1), lambda i: (0, 0))],
        out_specs=pl.BlockSpec((w_col.shape[0], blk), lambda i: (0, i)),
        out_shape=jax.ShapeDtypeStruct((w_col.shape[0], e_total), jnp.float32),
    )(w_col)


def kernel(particles, weights, edges):
    n, p, d = particles.shape
    e_total = edges.shape[1]
    w = weights.shape[1]
    table_t = particles.transpose(1, 2, 0).reshape(p * d, n)
    idx = edges.astype(jnp.int32)
    out_t = _edge_diff_sc(table_t, idx[0], idx[1])
    ratios = out_t.reshape(p, d, e_total).transpose(2, 0, 1)
    w_col = weights[0, :].reshape(w, 1)
    rm_weights = _rm_weights_tc(w_col, e_total).transpose(1, 0)
    return ratios, rm_weights


# speedup vs baseline: 1.2311x; 1.0050x over previous
import dataclasses
import functools

import jax
import jax.numpy as jnp
from jax import lax
from jax.experimental import pallas as pl
from jax.experimental.pallas import tpu as pltpu
from jax.experimental.pallas import tpu_sc as plsc

_N_CORES = 2
_N_SUBCORES = 16
_COLS_PER_TEC = 8
_EDGE_CHUNK = 1280


def _edge_diff_sc(table_t, idx_i, idx_j):
    n_feat, n_nodes = table_t.shape
    e_total = idx_i.shape[0]
    per_core = e_total // _N_CORES
    n_chunks = per_core // _EDGE_CHUNK
    n_pad = 2 * ((n_chunks + 1) // 2)
    mesh = plsc.VectorSubcoreMesh(core_axis_name="c", subcore_axis_name="s")
    cp = pltpu.CompilerParams()
    if "needs_layout_passes" in pltpu.CompilerParams.__dataclass_fields__:
        cp = dataclasses.replace(cp, needs_layout_passes=False)

    @functools.partial(
        pl.kernel,
        mesh=mesh,
        compiler_params=cp,
        out_type=jax.ShapeDtypeStruct((n_feat, e_total), jnp.float32),
        scratch_types=(
            [pltpu.VMEM((_COLS_PER_TEC, n_nodes), jnp.float32)]
            + [pltpu.VMEM((_EDGE_CHUNK,), jnp.int32) for _ in range(4)]
            + [pltpu.VMEM((_COLS_PER_TEC, _EDGE_CHUNK), jnp.float32) for _ in range(2)]
            + [pltpu.SemaphoreType.DMA for _ in range(6)]
        ),
    )
    def k(tab_hbm, ii_hbm, jj_hbm, out_hbm, tabv, *scratch):
        ii = scratch[0:2]
        jj = scratch[2:4]
        ov = scratch[4:6]
        sii = scratch[6:8]
        sjj = scratch[8:10]
        sw = scratch[10:12]
        core = lax.axis_index("c")
        sub = lax.axis_index("s")
        ebase = core * per_core
        cbase = sub * _COLS_PER_TEC
        pltpu.sync_copy(tab_hbm.at[pl.ds(cbase, _COLS_PER_TEC)], tabv)
        rowsel = [jnp.full((16,), cc, jnp.int32) for cc in range(_COLS_PER_TEC)]

        def issue_idx(c, par):
            off = ebase + c * _EDGE_CHUNK
            pltpu.async_copy(ii_hbm.at[pl.ds(off, _EDGE_CHUNK)], ii[par], sii[par])
            pltpu.async_copy(jj_hbm.at[pl.ds(off, _EDGE_CHUNK)], jj[par], sjj[par])

        def wait_idx(par):
            pltpu.make_async_copy(
                ii_hbm.at[pl.ds(0, _EDGE_CHUNK)], ii[par], sii[par]).wait()
            pltpu.make_async_copy(
                jj_hbm.at[pl.ds(0, _EDGE_CHUNK)], jj[par], sjj[par]).wait()

        def issue_write(c, par):
            pltpu.async_copy(
                ov[par],
                out_hbm.at[pl.ds(cbase, _COLS_PER_TEC),
                           pl.ds(ebase + c * _EDGE_CHUNK, _EDGE_CHUNK)],
                sw[par])

        def wait_write(par):
            pltpu.make_async_copy(
                ov[par],
                out_hbm.at[pl.ds(0, _COLS_PER_TEC), pl.ds(0, _EDGE_CHUNK)],
                sw[par]).wait()

        issue_idx(0, 0)

        @pl.loop(0, n_pad // 2)
        def _(t):
            for par in (0, 1):
                c = t * 2 + par

                @pl.when(c + 1 < n_chunks)
                def _():
                    issue_idx(c + 1, 1 - par)

                @pl.when(jnp.logical_and(c >= 2, c - 2 < n_chunks))
                def _():
                    wait_write(par)

                @pl.when(c < n_chunks)
                def _():
                    wait_idx(par)

                    @plsc.parallel_loop(0, _EDGE_CHUNK // 16, unroll=4)
                    def _(s):
                        sl = pl.ds(s * 16, 16)
                        i16 = ii[par][sl]
                        j16 = jj[par][sl]
                        for cc in range(_COLS_PER_TEC):
                            vi = plsc.load_gather(tabv, [rowsel[cc], i16])
                            vj = plsc.load_gather(tabv, [rowsel[cc], j16])
                            ov[par][cc, sl] = vi - vj

                    issue_write(c, par)

        if n_pad - 2 < n_chunks:
            wait_write((n_pad - 2) % 2)
        if n_pad - 1 < n_chunks:
            wait_write((n_pad - 1) % 2)

    return k(table_t, idx_i, idx_j)


def _fill_body(w_ref, o_ref):
    o_ref[...] = jnp.broadcast_to(w_ref[...], o_ref.shape)


def _rm_weights_tc(w_col, e_total):
    blk = 2560
    return pl.pallas_call(
        _fill_body,
        grid=(e_total // blk,),
        in_specs=[pl.BlockSpec((w_col.shape[0], 1), lambda i: (0, 0))],
        out_specs=pl.BlockSpec((w_col.shape[0], blk), lambda i: (0, i)),
        out_shape=jax.ShapeDtypeStruct((w_col.shape[0], e_total), jnp.float32),
    )(w_col)


def kernel(particles, weights, edges):
    n, p, d = particles.shape
    e_total = edges.shape[1]
    w = weights.shape[1]
    table_t = particles.transpose(1, 2, 0).reshape(p * d, n)
    idx = edges.astype(jnp.int32)
    out_t = _edge_diff_sc(table_t, idx[0], idx[1])
    ratios = out_t.reshape(p, d, e_total).transpose(2, 0, 1)
    w_col = weights[0, :].reshape(w, 1)
    rm_weights = _rm_weights_tc(w_col, e_total).transpose(1, 0)
    return ratios, rm_weights


# speedup vs baseline: 1.2983x; 1.0546x over previous
import dataclasses
import functools

import jax
import jax.numpy as jnp
from jax import lax
from jax.experimental import pallas as pl
from jax.experimental.pallas import tpu as pltpu
from jax.experimental.pallas import tpu_sc as plsc

_N_CORES = 2
_N_SUBCORES = 16
_COLS_PER_TEC = 8
_EDGE_CHUNK = 1280


def _edge_diff_sc(table_t, idx_flat):
    n_feat, n_nodes = table_t.shape
    e_total = idx_flat.shape[0] // 2
    per_core = e_total // _N_CORES
    n_chunks = per_core // _EDGE_CHUNK
    n_pad = 2 * ((n_chunks + 1) // 2)
    mesh = plsc.VectorSubcoreMesh(core_axis_name="c", subcore_axis_name="s")
    cp = pltpu.CompilerParams()
    if "needs_layout_passes" in pltpu.CompilerParams.__dataclass_fields__:
        cp = dataclasses.replace(cp, needs_layout_passes=False)

    @functools.partial(
        pl.kernel,
        mesh=mesh,
        compiler_params=cp,
        out_type=jax.ShapeDtypeStruct((n_feat, e_total), jnp.float32),
        scratch_types=(
            [pltpu.VMEM((_COLS_PER_TEC, n_nodes), jnp.float32)]
            + [pltpu.VMEM((_EDGE_CHUNK,), jnp.int32) for _ in range(4)]
            + [pltpu.VMEM((_COLS_PER_TEC, _EDGE_CHUNK), jnp.float32) for _ in range(2)]
            + [pltpu.SemaphoreType.DMA for _ in range(6)]
        ),
    )
    def k(tab_hbm, idx_hbm, out_hbm, tabv, *scratch):
        ii = scratch[0:2]
        jj = scratch[2:4]
        ov = scratch[4:6]
        sii = scratch[6:8]
        sjj = scratch[8:10]
        sw = scratch[10:12]
        core = lax.axis_index("c")
        sub = lax.axis_index("s")
        ebase = core * per_core
        cbase = sub * _COLS_PER_TEC
        pltpu.sync_copy(tab_hbm.at[pl.ds(cbase, _COLS_PER_TEC)], tabv)
        rowsel = [jnp.full((16,), cc, jnp.int32) for cc in range(_COLS_PER_TEC)]

        def issue_idx(c, par):
            off = ebase + c * _EDGE_CHUNK
            pltpu.async_copy(idx_hbm.at[pl.ds(off, _EDGE_CHUNK)], ii[par], sii[par])
            pltpu.async_copy(
                idx_hbm.at[pl.ds(e_total + off, _EDGE_CHUNK)], jj[par], sjj[par])

        def wait_idx(par):
            pltpu.make_async_copy(
                idx_hbm.at[pl.ds(0, _EDGE_CHUNK)], ii[par], sii[par]).wait()
            pltpu.make_async_copy(
                idx_hbm.at[pl.ds(0, _EDGE_CHUNK)], jj[par], sjj[par]).wait()

        def issue_write(c, par):
            pltpu.async_copy(
                ov[par],
                out_hbm.at[pl.ds(cbase, _COLS_PER_TEC),
                           pl.ds(ebase + c * _EDGE_CHUNK, _EDGE_CHUNK)],
                sw[par])

        def wait_write(par):
            pltpu.make_async_copy(
                ov[par],
                out_hbm.at[pl.ds(0, _COLS_PER_TEC), pl.ds(0, _EDGE_CHUNK)],
                sw[par]).wait()

        issue_idx(0, 0)

        @pl.loop(0, n_pad // 2)
        def _(t):
            for par in (0, 1):
                c = t * 2 + par

                @pl.when(c + 1 < n_chunks)
                def _():
                    issue_idx(c + 1, 1 - par)

                @pl.when(jnp.logical_and(c >= 2, c - 2 < n_chunks))
                def _():
                    wait_write(par)

                @pl.when(c < n_chunks)
                def _():
                    wait_idx(par)

                    @plsc.parallel_loop(0, _EDGE_CHUNK // 16, unroll=4)
                    def _(s):
                        sl = pl.ds(s * 16, 16)
                        i16 = ii[par][sl]
                        j16 = jj[par][sl]
                        for cc in range(_COLS_PER_TEC):
                            vi = plsc.load_gather(tabv, [rowsel[cc], i16])
                            vj = plsc.load_gather(tabv, [rowsel[cc], j16])
                            ov[par][cc, sl] = vi - vj

                    issue_write(c, par)

        if n_pad - 2 < n_chunks:
            wait_write((n_pad - 2) % 2)
        if n_pad - 1 < n_chunks:
            wait_write((n_pad - 1) % 2)

    return k(table_t, idx_flat)


def _fill_body(w_ref, o_ref):
    o_ref[...] = jnp.broadcast_to(w_ref[...], o_ref.shape)


def _rm_weights_tc(w_col, e_total):
    blk = 2560
    return pl.pallas_call(
        _fill_body,
        grid=(e_total // blk,),
        in_specs=[pl.BlockSpec((w_col.shape[0], 1), lambda i: (0, 0))],
        out_specs=pl.BlockSpec((w_col.shape[0], blk), lambda i: (0, i)),
        out_shape=jax.ShapeDtypeStruct((w_col.shape[0], e_total), jnp.float32),
    )(w_col)


def kernel(particles, weights, edges):
    n, p, d = particles.shape
    e_total = edges.shape[1]
    w = weights.shape[1]
    table_t = particles.transpose(1, 2, 0).reshape(p * d, n)
    idx_flat = edges.astype(jnp.int32).reshape(2 * e_total)
    out_t = _edge_diff_sc(table_t, idx_flat)
    ratios = out_t.reshape(p, d, e_total).transpose(2, 0, 1)
    w_col = weights[0, :].reshape(w, 1)
    rm_weights = _rm_weights_tc(w_col, e_total).transpose(1, 0)
    return ratios, rm_weights
